# baseline (speedup 1.0000x reference)
import functools
import jax
import jax.numpy as jnp
import numpy as np
from jax import lax
from jax.experimental import pallas as pl
from jax.experimental.pallas import tpu as pltpu
from jax.experimental.pallas import tpu_sc as plsc

EMBED = 64
DM = 4096
OFF = (0, 16, 24, 48)
KP = 128
CLIP_HI = (12, 6, 23, 31)

ROWS_BLK = 1024
NC, NS = 2, 16
NW = NC * NS
LANES = 16


def _proj_kernel(z_ref, w_ref, b_ref, p_ref):
    zw = lax.dot_general(
        z_ref[...], w_ref[...], (((1,), (1,)), ((), ())),
        preferred_element_type=jnp.float32)
    p = zw * 0.5 + 0.125 * b_ref[...]
    p_ref[...] = p.astype(jnp.bfloat16)


def _mm_kernel(mh_ref, p_ref, out_ref):
    out_ref[...] = lax.dot_general(
        mh_ref[...].astype(jnp.bfloat16), p_ref[...],
        (((1,), (0,)), ((), ())),
        preferred_element_type=jnp.float32)


def _make_multihot_sc(n_rows):
    chunk = n_rows // NW
    half = chunk // 2
    groups = half // LANES
    mesh = plsc.VectorSubcoreMesh(
        core_axis_name="c", subcore_axis_name="s",
        num_cores=NC, num_subcores=NS)

    @functools.partial(
        pl.kernel,
        out_type=jax.ShapeDtypeStruct((n_rows, KP), jnp.float32),
        mesh=mesh,
        scratch_types=[
            pltpu.VMEM((4 * chunk,), jnp.int32),
            pltpu.VMEM((half, KP), jnp.float32),
            pltpu.SemaphoreType.DMA,
        ],
        compiler_params=pltpu.CompilerParams(needs_layout_passes=False),
    )
    def mh_kernel(idx_hbm, mh_hbm, idx_v, m_v, sem):
        wid = lax.axis_index("s") * NC + lax.axis_index("c")
        base = wid * chunk
        copy = pltpu.async_copy(idx_hbm.at[pl.ds(base * 4, chunk * 4)],
                                idx_v, sem)

        zeros = jnp.zeros((LANES,), jnp.float32)
        ones = jnp.ones((LANES,), jnp.float32)
        lane = lax.iota(jnp.int32, LANES)

        def zbody(i, carry):
            for j in range(KP // LANES):
                m_v[i, pl.ds(j * LANES, LANES)] = zeros
            return carry

        for h in range(2):
            lax.fori_loop(0, half, zbody, 0)
            if h == 0:
                copy.wait()

            def sbody(g, carry):
                rows = g * LANES + lane
                gbase = (h * half + g * LANES) * 4 + lane * 4
                for t in range(4):
                    iv = plsc.load_gather(idx_v, [gbase + t])
                    iv = jnp.clip(iv, 0, CLIP_HI[t])
                    plsc.store_scatter(m_v, [rows, OFF[t] + iv], ones)
                return carry

            lax.fori_loop(0, groups, sbody, 0)
            pltpu.sync_copy(m_v, mh_hbm.at[pl.ds(base + h * half, half)])

    return mh_kernel


def kernel(time_feats, month_w, weekday_w, hour_w, day_w, W, b):
    B, S, F = time_feats.shape
    N = B * S
    idx_flat = time_feats.astype(jnp.int32).reshape(N * F)

    mh = _make_multihot_sc(N)(idx_flat)

    z = jnp.zeros((KP, 4 * EMBED), jnp.float32)
    for t, tbl in enumerate((month_w, weekday_w, hour_w, day_w)):
        z = lax.dynamic_update_slice(z, tbl, (OFF[t], t * EMBED))

    p = pl.pallas_call(
        _proj_kernel,
        out_shape=jax.ShapeDtypeStruct((KP, DM), jnp.bfloat16),
    )(z, W, b.reshape(1, DM))

    nblk = N // ROWS_BLK
    out = pl.pallas_call(
        _mm_kernel,
        grid=(nblk,),
        in_specs=[
            pl.BlockSpec((ROWS_BLK, KP), lambda i: (i, 0)),
            pl.BlockSpec((KP, DM), lambda i: (0, 0)),
        ],
        out_specs=pl.BlockSpec((ROWS_BLK, DM), lambda i: (i, 0)),
        out_shape=jax.ShapeDtypeStruct((N, DM), jnp.float32),
    )(mh, p)
    return out.reshape(B, S, DM)


# speedup vs baseline: 1.1538x; 1.1538x over previous
import functools
import jax
import jax.numpy as jnp
import numpy as np
from jax import lax
from jax.experimental import pallas as pl
from jax.experimental.pallas import tpu as pltpu
from jax.experimental.pallas import tpu_sc as plsc

EMBED = 64
DM = 4096
OFF = (0, 16, 24, 48)
KP = 128
CLIP_HI = (12, 6, 23, 31)

ROWS_BLK = 1024
NC, NS = 2, 16
NW = NC * NS
LANES = 16


def _proj_kernel(z_ref, w_ref, b_ref, p_ref):
    zw = lax.dot_general(
        z_ref[...], w_ref[...], (((1,), (1,)), ((), ())),
        preferred_element_type=jnp.float32)
    p = zw * 0.5 + 0.125 * b_ref[...]
    p_ref[...] = p.astype(jnp.bfloat16)


def _mm_kernel(mh_ref, p_ref, out_ref):
    out_ref[...] = lax.dot_general(
        mh_ref[...].astype(jnp.bfloat16), p_ref[...],
        (((1,), (0,)), ((), ())),
        preferred_element_type=jnp.float32)


def _make_multihot_sc(n_rows):
    chunk = n_rows // NW
    quarter = chunk // 4
    groups = quarter // LANES
    mesh = plsc.VectorSubcoreMesh(
        core_axis_name="c", subcore_axis_name="s",
        num_cores=NC, num_subcores=NS)

    @functools.partial(
        pl.kernel,
        out_type=jax.ShapeDtypeStruct((n_rows, KP), jnp.float32),
        mesh=mesh,
        scratch_types=[
            pltpu.VMEM((4 * chunk,), jnp.int32),
            pltpu.VMEM((2, quarter, KP), jnp.float32),
            pltpu.SemaphoreType.DMA,
            pltpu.SemaphoreType.DMA,
            pltpu.SemaphoreType.DMA,
        ],
        compiler_params=pltpu.CompilerParams(needs_layout_passes=False),
    )
    def mh_kernel(idx_hbm, mh_hbm, idx_v, m_v, isem, sem0, sem1):
        wid = lax.axis_index("s") * NC + lax.axis_index("c")
        base = wid * chunk
        copies = [
            pltpu.async_copy(idx_hbm.at[pl.ds(t * n_rows + base, chunk)],
                             idx_v.at[pl.ds(t * chunk, chunk)], isem)
            for t in range(4)
        ]

        zeros = jnp.zeros((LANES,), jnp.float32)
        ones = jnp.ones((LANES,), jnp.float32)
        lane = lax.iota(jnp.int32, LANES)
        wsems = (sem0, sem1)

        def zbody(i, carry):
            for j in range(KP // LANES):
                m_v[i // quarter, i % quarter, pl.ds(j * LANES, LANES)] = zeros
            return carry

        lax.fori_loop(0, 2 * quarter, zbody, 0)
        for c in copies:
            c.wait()

        def scatter_q(q, bufi, val):
            def sbody(g, carry):
                rows = g * LANES + lane
                for t in range(4):
                    iv = idx_v[pl.ds(t * chunk + q * quarter + g * LANES,
                                     LANES)]
                    iv = jnp.clip(iv, 0, CLIP_HI[t])
                    plsc.store_scatter(m_v.at[bufi], [rows, OFF[t] + iv], val)
                return carry
            lax.fori_loop(0, groups, sbody, 0)

        wb = [None, None]
        for q in range(4):
            bufi = q % 2
            if wb[bufi] is not None:
                wb[bufi].wait()
                scatter_q(q - 2, bufi, zeros)
            scatter_q(q, bufi, ones)
            wb[bufi] = pltpu.async_copy(
                m_v.at[bufi],
                mh_hbm.at[pl.ds(base + q * quarter, quarter)],
                wsems[bufi])
        wb[0].wait()
        wb[1].wait()

    return mh_kernel


def kernel(time_feats, month_w, weekday_w, hour_w, day_w, W, b):
    B, S, F = time_feats.shape
    N = B * S
    idx_t = time_feats.reshape(N, F).astype(jnp.int32).T.reshape(-1)

    mh = _make_multihot_sc(N)(idx_t)

    z = jnp.zeros((KP, 4 * EMBED), jnp.float32)
    for t, tbl in enumerate((month_w, weekday_w, hour_w, day_w)):
        z = lax.dynamic_update_slice(z, tbl, (OFF[t], t * EMBED))

    p = pl.pallas_call(
        _proj_kernel,
        out_shape=jax.ShapeDtypeStruct((KP, DM), jnp.bfloat16),
    )(z, W, b.reshape(1, DM))

    nblk = N // ROWS_BLK
    out = pl.pallas_call(
        _mm_kernel,
        grid=(nblk,),
        in_specs=[
            pl.BlockSpec((ROWS_BLK, KP), lambda i: (i, 0)),
            pl.BlockSpec((KP, DM), lambda i: (0, 0)),
        ],
        out_specs=pl.BlockSpec((ROWS_BLK, DM), lambda i: (i, 0)),
        out_shape=jax.ShapeDtypeStruct((N, DM), jnp.float32),
    )(mh, p)
    return out.reshape(B, S, DM)
